# Initial kernel scaffold; baseline (speedup 1.0000x reference)
import jax
import jax.numpy as jnp
from jax.experimental import pallas as pl


def kernel(x, row_index, col_index, to_col_index, att_bias, dist, pos, col_pos, Wq, bq, Wk, bk):
    raise NotImplementedError("write your pallas kernel here")


# speedup vs baseline: 28.8204x; 28.8204x over previous
import functools

import jax
import jax.numpy as jnp
from jax import lax
from jax.experimental import pallas as pl
from jax.experimental.pallas import tpu as pltpu
from jax.experimental.pallas import tpu_sc as plsc

N = 10000
E = 320000
M = 10000
D = 128
H = 8
DH = 16

NW = 32
RPW = 313
N_PAD = 10240
E_BLK = 512
K_BLK = 64
E_PAD = E + 2 * E_BLK
ACC_W = 40
ACC_PAD = 12528


def _proj_body(x_ref, wq_ref, bq_ref, wk_ref, bk_ref, q_ref, kf_ref):
    xb = x_ref[...]
    q = jnp.dot(xb, wq_ref[...], preferred_element_type=jnp.float32)
    q_ref[...] = (q + bq_ref[...]) * 0.25
    kf = jnp.dot(xb, wk_ref[...], preferred_element_type=jnp.float32)
    kf_ref[...] = kf + bk_ref[...]


def _project(x_pad, Wq, bq, Wk, bk):
    grid = N_PAD // 256
    return pl.pallas_call(
        _proj_body,
        grid=(grid,),
        in_specs=[
            pl.BlockSpec((256, D), lambda i: (i, 0)),
            pl.BlockSpec((D, D), lambda i: (0, 0)),
            pl.BlockSpec((1, D), lambda i: (0, 0)),
            pl.BlockSpec((D, D), lambda i: (0, 0)),
            pl.BlockSpec((1, D), lambda i: (0, 0)),
        ],
        out_specs=[
            pl.BlockSpec((256, D), lambda i: (i, 0)),
            pl.BlockSpec((256, D), lambda i: (i, 0)),
        ],
        out_shape=[
            jax.ShapeDtypeStruct((N_PAD, D), jnp.float32),
            jax.ShapeDtypeStruct((N_PAD, D), jnp.float32),
        ],
    )(x_pad, Wq, bq.reshape(1, D), Wk, bk.reshape(1, D))


def _edge_body(q_hbm, kf_hbm, row_hbm, col_hbm, bias_hbm, dist_hbm,
               tocol_hbm, cpos_hbm, rp_hbm, acc_hbm,
               qloc, tocol_v, cpos_v, rp_v, acc_v,
               row_v, col_v, bias_v, dist_v, ci2_v, kbuf, sh_v, sb_v, sem):
    wid = lax.axis_index("s") * 2 + lax.axis_index("c")
    rbase = wid * RPW
    iota = lax.iota(jnp.int32, 16)

    pltpu.sync_copy(rp_hbm, rp_v)
    pltpu.sync_copy(tocol_hbm, tocol_v)
    pltpu.sync_copy(cpos_hbm, cpos_v)
    pltpu.sync_copy(q_hbm.at[pl.ds(pl.multiple_of(rbase * D, 64), RPW * D)], qloc)

    v0 = rp_v[pl.ds(0, 16)]
    v1 = rp_v[pl.ds(16, 16)]
    v2 = rp_v[pl.ds(32, 16)]

    def _ext(vec, j):
        picked = jnp.where(iota == j, vec.astype(jnp.float32), 0.0)
        return jnp.sum(picked).astype(jnp.int32)

    e_start = jnp.where(wid < 16, _ext(v0, wid), _ext(v1, wid - 16))
    w1 = wid + 1
    e_end = jnp.where(
        w1 < 16, _ext(v0, w1),
        jnp.where(w1 < 32, _ext(v1, w1 - 16), _ext(v2, w1 - 32)))

    zeros16 = jnp.zeros((16,), jnp.float32)

    def _z(i, c):
        acc_v[pl.ds(pl.multiple_of(i * 16, 16), 16)] = zeros16
        return c

    lax.fori_loop(0, ACC_PAD // 16, _z, 0)

    estart8 = pl.multiple_of((e_start // 8) * 8, 8)
    nblk = (e_end - estart8 + (E_BLK - 1)) // E_BLK

    def _group(bstart, off, g):
        row16 = row_v[pl.ds(off, 16)]
        col16 = col_v[pl.ds(off, 16)]
        bias16 = bias_v[pl.ds(off, 16)]
        dist16 = dist_v[pl.ds(off, 16)]
        eidx = (bstart + off) + iota
        emask = (eidx >= e_start) & (eidx < e_end)
        lrow = jnp.clip(row16 - rbase, 0, RPW - 1)
        biasm = jnp.where(emask, bias16, -100.0)
        qbase = lrow * D
        koff = iota + g * 16

        invd = jnp.where(dist16 == 0.0, 0.0, 1.0 / dist16)
        col3 = col16 * 3
        cpx = plsc.load_gather(cpos_v, [col3])
        cpy = plsc.load_gather(cpos_v, [col3 + 1])
        cpz = plsc.load_gather(cpos_v, [col3 + 2])

        sh_v[pl.ds(1, 16)] = lrow
        prv = sh_v[pl.ds(0, 16)]
        nxt = sh_v[pl.ds(2, 16)]
        is_first = (lrow != prv) | (iota == 0)
        is_last = (lrow != nxt) | (iota == 15)
        f = plsc.cummax(jnp.where(is_first, iota, 0))
        fm1 = jnp.maximum(f - 1, 0)
        hasp = f > 0
        lrow40 = lrow * ACC_W

        def _seg_add(v, offc):
            b = plsc.cumsum(v)
            sb_v[pl.ds(0, 16)] = b
            pb = plsc.load_gather(sb_v, [fm1])
            tot = b - jnp.where(hasp, pb, 0.0)
            plsc.addupdate_scatter(acc_v, [lrow40 + offc], tot, mask=is_last)

        for h in range(H):
            lg = biasm
            for d in range(DH):
                c = h * DH + d
                qv = plsc.load_gather(qloc, [qbase + c])
                kv = plsc.load_gather(kbuf, [koff, jnp.full((16,), c, jnp.int32)])
                lg = lg + qv * kv
            p = jnp.exp(lg)
            w = p * invd
            _seg_add(p, h)
            _seg_add(w, 8 + h)
            _seg_add(w * cpx, 16 + h)
            _seg_add(w * cpy, 24 + h)
            _seg_add(w * cpz, 32 + h)

    def _blk(b, carry):
        bstart = pl.multiple_of(estart8 + b * E_BLK, 8)
        pltpu.sync_copy(row_hbm.at[pl.ds(bstart, E_BLK)], row_v)
        pltpu.sync_copy(col_hbm.at[pl.ds(bstart, E_BLK)], col_v)
        pltpu.sync_copy(bias_hbm.at[pl.ds(bstart, E_BLK)], bias_v)
        pltpu.sync_copy(dist_hbm.at[pl.ds(bstart, E_BLK)], dist_v)

        def _sub(k, c2):
            off0 = k * K_BLK
            for g in range(K_BLK // 16):
                off = pl.multiple_of(off0 + g * 16, 16)
                col16 = col_v[pl.ds(off, 16)]
                ci2 = plsc.load_gather(tocol_v, [col16])
                ci2_v[pl.ds(g * 16, 16)] = ci2
            pltpu.async_copy(kf_hbm.at[ci2_v], kbuf, sem).wait()
            for g in range(K_BLK // 16):
                off = pl.multiple_of(off0 + g * 16, 16)
                _group(bstart, off, g)
            return c2

        lax.fori_loop(0, E_BLK // K_BLK, _sub, 0)
        return carry

    lax.fori_loop(0, nblk, _blk, 0)
    pltpu.sync_copy(acc_v, acc_hbm.at[wid])


def _edge_pass(q, kf, row_e, col_e, bias_e, dist_e, tocol, cpos_flat, rp):
    mesh = plsc.VectorSubcoreMesh(core_axis_name="c", subcore_axis_name="s")
    f = pl.kernel(
        _edge_body,
        mesh=mesh,
        out_type=jax.ShapeDtypeStruct((NW, ACC_PAD), jnp.float32),
        scratch_types=[
            pltpu.VMEM((RPW * D,), jnp.float32),
            pltpu.VMEM((M,), jnp.int32),
            pltpu.VMEM((3 * M,), jnp.float32),
            pltpu.VMEM((48,), jnp.int32),
            pltpu.VMEM((ACC_PAD,), jnp.float32),
            pltpu.VMEM((E_BLK,), jnp.int32),
            pltpu.VMEM((E_BLK,), jnp.int32),
            pltpu.VMEM((E_BLK,), jnp.float32),
            pltpu.VMEM((E_BLK,), jnp.float32),
            pltpu.VMEM((K_BLK,), jnp.int32),
            pltpu.VMEM((K_BLK, D), jnp.float32),
            pltpu.VMEM((32,), jnp.int32),
            pltpu.VMEM((16,), jnp.float32),
            pltpu.SemaphoreType.DMA,
        ],
        compiler_params=pltpu.CompilerParams(needs_layout_passes=False),
    )
    return f(q.reshape(-1), kf, row_e, col_e, bias_e, dist_e,
             tocol, cpos_flat, rp)


def _feat_body(acc_ref, pos_ref, fx_ref, fy_ref, fz_ref, av_ref):
    a = acc_ref[...]
    s = a[0:8, :]
    A = a[8:16, :]
    Dx = a[16:24, :]
    Dy = a[24:32, :]
    Dz = a[32:40, :]
    inv = jnp.where(s > 0.0, 1.0 / s, 0.0)
    avg = A * inv
    px = pos_ref[0:1, :]
    py = pos_ref[1:2, :]
    pz = pos_ref[2:3, :]
    dx = Dx * inv - avg * px
    dy = Dy * inv - avg * py
    dz = Dz * inv - avg * pz
    nrm = jnp.sqrt(dx * dx + dy * dy + dz * dz)
    sc = 1.0 / jnp.maximum(nrm, 1e-12)
    fx_ref[...] = dx * sc
    fy_ref[...] = dy * sc
    fz_ref[...] = dz * sc
    av_ref[...] = avg


def _featurize(acc_t, pos_t):
    B = 512
    grid = (N + B - 1) // B
    out = jax.ShapeDtypeStruct((H, N), jnp.float32)
    return pl.pallas_call(
        _feat_body,
        grid=(grid,),
        in_specs=[
            pl.BlockSpec((ACC_W, B), lambda i: (0, i)),
            pl.BlockSpec((3, B), lambda i: (0, i)),
        ],
        out_specs=[pl.BlockSpec((H, B), lambda i: (0, i))] * 4,
        out_shape=[out, out, out, out],
    )(acc_t, pos_t)


def kernel(x, row_index, col_index, to_col_index, att_bias, dist, pos,
           col_pos, Wq, bq, Wk, bk):
    x_pad = jnp.pad(x, ((0, N_PAD - N), (0, 0)))
    q, kf = _project(x_pad, Wq, bq, Wk, bk)

    row_i = row_index.astype(jnp.int32)
    rp = jnp.searchsorted(
        row_i, jnp.arange(33, dtype=jnp.int32) * RPW, side="left"
    ).astype(jnp.int32)
    rp = jnp.pad(rp, (0, 48 - 33), constant_values=E)

    row_e = jnp.pad(row_i, (0, E_PAD - E))
    col_e = jnp.pad(col_index.astype(jnp.int32), (0, E_PAD - E))
    bias_e = jnp.pad(att_bias, (0, E_PAD - E))
    dist_e = jnp.pad(dist, (0, E_PAD - E), constant_values=1.0)

    acc = _edge_pass(q, kf, row_e, col_e, bias_e, dist_e,
                     to_col_index.astype(jnp.int32),
                     col_pos.reshape(-1), rp)

    acc_t = acc[:, :RPW * ACC_W].reshape(NW * RPW, ACC_W)[:N].T
    fx, fy, fz, av = _featurize(acc_t, pos.T)

    feat = jnp.stack([fx, fy, fz, av], axis=-1)
    return feat.transpose(1, 0, 2).reshape(N, H * 4)


# speedup vs baseline: 31.9335x; 1.1080x over previous
import jax
import jax.numpy as jnp
from jax import lax
from jax.experimental import pallas as pl
from jax.experimental.pallas import tpu as pltpu
from jax.experimental.pallas import tpu_sc as plsc

N = 10000
E = 320000
M = 10000
D = 128
H = 8
DH = 16

NW = 32
RPW = 313
N_PAD = 10240
E_BLK = 512
K_BLK = 64
SPB = E_BLK // K_BLK
E_PAD = E + 2 * E_BLK
QROWS = 320
ACC_W = 40
ACC_PAD = 12528


def _proj_body(x_ref, wq_ref, bq_ref, wk_ref, bk_ref, q_ref, kf_ref):
    xb = x_ref[...]
    q = jnp.dot(xb, wq_ref[...], preferred_element_type=jnp.float32)
    q_ref[...] = (q + bq_ref[...]) * 0.25
    kf = jnp.dot(xb, wk_ref[...], preferred_element_type=jnp.float32)
    kf_ref[...] = kf + bk_ref[...]


def _project(x_pad, Wq, bq, Wk, bk):
    grid = N_PAD // 256
    return pl.pallas_call(
        _proj_body,
        grid=(grid,),
        in_specs=[
            pl.BlockSpec((256, D), lambda i: (i, 0)),
            pl.BlockSpec((D, D), lambda i: (0, 0)),
            pl.BlockSpec((1, D), lambda i: (0, 0)),
            pl.BlockSpec((D, D), lambda i: (0, 0)),
            pl.BlockSpec((1, D), lambda i: (0, 0)),
        ],
        out_specs=[
            pl.BlockSpec((256, D), lambda i: (i, 0)),
            pl.BlockSpec((256, D), lambda i: (i, 0)),
        ],
        out_shape=[
            jax.ShapeDtypeStruct((N_PAD, D), jnp.float32),
            jax.ShapeDtypeStruct((N_PAD, D), jnp.float32),
        ],
    )(x_pad, Wq, bq.reshape(1, D), Wk, bk.reshape(1, D))


def _edge_body(q_hbm, kf_hbm, row_hbm, col_hbm, bias_hbm, dist_hbm,
               tocol_hbm, cpos_hbm, rp_hbm, acc_hbm,
               qloc, tocol_v, cpos_v, rp_v, acc_v,
               row_v, col_v, bias_v, dist_v, ci2_v, kbuf, sh_v, sb_v,
               ksem, esem):
    wid = lax.axis_index("s") * 2 + lax.axis_index("c")
    rbase = wid * RPW
    iota = lax.iota(jnp.int32, 16)

    pltpu.sync_copy(rp_hbm, rp_v)
    pltpu.sync_copy(tocol_hbm, tocol_v)
    pltpu.sync_copy(cpos_hbm, cpos_v)
    rb8 = pl.multiple_of((rbase // 8) * 8, 8)
    rdelta = rbase - rb8
    pltpu.sync_copy(q_hbm.at[pl.ds(rb8, QROWS)], qloc)

    v0 = rp_v[pl.ds(0, 16)]
    v1 = rp_v[pl.ds(16, 16)]
    v2 = rp_v[pl.ds(32, 16)]

    def _ext(vec, j):
        picked = jnp.where(iota == j, vec.astype(jnp.float32), 0.0)
        return jnp.sum(picked).astype(jnp.int32)

    e_start = jnp.where(wid < 16, _ext(v0, wid), _ext(v1, wid - 16))
    w1 = wid + 1
    e_end = jnp.where(
        w1 < 16, _ext(v0, w1),
        jnp.where(w1 < 32, _ext(v1, w1 - 16), _ext(v2, w1 - 32)))

    zeros16 = jnp.zeros((16,), jnp.float32)

    def _z(i, c):
        acc_v[pl.ds(pl.multiple_of(i * 16, 16), 16)] = zeros16
        return c

    lax.fori_loop(0, ACC_PAD // 16, _z, 0)

    estart8 = pl.multiple_of((e_start // 8) * 8, 8)
    nsub = (e_end - estart8 + (K_BLK - 1)) // K_BLK
    nblk = (e_end - estart8 + (E_BLK - 1)) // E_BLK

    def _stage_start(b):
        bstart = pl.multiple_of(estart8 + b * E_BLK, 8)
        half = pl.multiple_of((b % 2) * E_BLK, 8)
        pltpu.async_copy(row_hbm.at[pl.ds(bstart, E_BLK)],
                         row_v.at[pl.ds(half, E_BLK)], esem)
        pltpu.async_copy(col_hbm.at[pl.ds(bstart, E_BLK)],
                         col_v.at[pl.ds(half, E_BLK)], esem)
        pltpu.async_copy(bias_hbm.at[pl.ds(bstart, E_BLK)],
                         bias_v.at[pl.ds(half, E_BLK)], esem)
        pltpu.async_copy(dist_hbm.at[pl.ds(bstart, E_BLK)],
                         dist_v.at[pl.ds(half, E_BLK)], esem)

    def _stage_wait(b):
        bstart = pl.multiple_of(estart8 + b * E_BLK, 8)
        half = pl.multiple_of((b % 2) * E_BLK, 8)
        pltpu.make_async_copy(row_hbm.at[pl.ds(bstart, E_BLK)],
                              row_v.at[pl.ds(half, E_BLK)], esem).wait()
        pltpu.make_async_copy(col_hbm.at[pl.ds(bstart, E_BLK)],
                              col_v.at[pl.ds(half, E_BLK)], esem).wait()
        pltpu.make_async_copy(bias_hbm.at[pl.ds(bstart, E_BLK)],
                              bias_v.at[pl.ds(half, E_BLK)], esem).wait()
        pltpu.make_async_copy(dist_hbm.at[pl.ds(bstart, E_BLK)],
                              dist_v.at[pl.ds(half, E_BLK)], esem).wait()

    def _kslices(s):
        khalf = pl.multiple_of((s % 2) * K_BLK, 8)
        return ci2_v.at[pl.ds(khalf, K_BLK)], kbuf.at[pl.ds(khalf, K_BLK)]

    def _ci2_and_fire(s):
        ehalf = ((s // SPB) % 2) * E_BLK
        ebase = ehalf + (s % SPB) * K_BLK
        khalf = (s % 2) * K_BLK
        for g in range(K_BLK // 16):
            off = pl.multiple_of(ebase + g * 16, 16)
            col16 = col_v[pl.ds(off, 16)]
            ci2 = plsc.load_gather(tocol_v, [col16])
            ci2_v[pl.ds(pl.multiple_of(khalf + g * 16, 16), 16)] = ci2
        ci_ref, kb_ref = _kslices(s)
        pltpu.async_copy(kf_hbm.at[ci_ref], kb_ref, ksem)

    def _kwait(s):
        ci_ref, kb_ref = _kslices(s)
        pltpu.make_async_copy(kf_hbm.at[ci_ref], kb_ref, ksem).wait()

    def _group(s, g):
        ehalf = ((s // SPB) % 2) * E_BLK
        ebase = ehalf + (s % SPB) * K_BLK
        off = pl.multiple_of(ebase + g * 16, 16)
        khalf = (s % 2) * K_BLK
        row16 = row_v[pl.ds(off, 16)]
        col16 = col_v[pl.ds(off, 16)]
        bias16 = bias_v[pl.ds(off, 16)]
        dist16 = dist_v[pl.ds(off, 16)]
        eidx = (estart8 + s * K_BLK + g * 16) + iota
        emask = (eidx >= e_start) & (eidx < e_end)
        lrow = jnp.clip(row16 - rbase, 0, RPW - 1)
        lrowq = lrow + rdelta
        biasm = jnp.where(emask, bias16, -100.0)
        koff = iota + (khalf + g * 16)

        invd = jnp.where(dist16 == 0.0, 0.0, 1.0 / dist16)
        col3 = col16 * 3
        cpx = plsc.load_gather(cpos_v, [col3])
        cpy = plsc.load_gather(cpos_v, [col3 + 1])
        cpz = plsc.load_gather(cpos_v, [col3 + 2])

        sh_v[pl.ds(1, 16)] = lrow
        prv = sh_v[pl.ds(0, 16)]
        nxt = sh_v[pl.ds(2, 16)]
        is_first = (lrow != prv) | (iota == 0)
        is_last = (lrow != nxt) | (iota == 15)
        f = plsc.cummax(jnp.where(is_first, iota, 0))
        fm1 = jnp.maximum(f - 1, 0)
        hasp = f > 0
        lrow40 = lrow * ACC_W

        def _seg_add(v, offc):
            b = plsc.cumsum(v)
            sb_v[pl.ds(0, 16)] = b
            pb = plsc.load_gather(sb_v, [fm1])
            tot = b - jnp.where(hasp, pb, 0.0)
            plsc.addupdate_scatter(acc_v, [lrow40 + offc], tot, mask=is_last)

        for h in range(H):
            lg = biasm
            for d in range(DH):
                c = h * DH + d
                cvec = jnp.full((16,), c, jnp.int32)
                qv = plsc.load_gather(qloc, [lrowq, cvec])
                kv = plsc.load_gather(kbuf, [koff, cvec])
                lg = lg + qv * kv
            p = jnp.exp(lg)
            w = p * invd
            _seg_add(p, h)
            _seg_add(w, 8 + h)
            _seg_add(w * cpx, 16 + h)
            _seg_add(w * cpy, 24 + h)
            _seg_add(w * cpz, 32 + h)

    @pl.when(nsub > 0)
    def _():
        _stage_start(0)
        _stage_wait(0)
        _ci2_and_fire(0)

    def _sub(s, carry):
        @pl.when((s % SPB == 0) & (s // SPB + 1 < nblk))
        def _():
            _stage_start(s // SPB + 1)

        @pl.when((s + 1 < nsub) & ((s + 1) % SPB != 0))
        def _():
            _ci2_and_fire(s + 1)

        _kwait(s)
        for g in range(K_BLK // 16):
            _group(s, g)

        @pl.when((s + 1 < nsub) & ((s + 1) % SPB == 0))
        def _():
            _stage_wait(s // SPB + 1)
            _ci2_and_fire(s + 1)

        return carry

    lax.fori_loop(0, nsub, _sub, 0)
    pltpu.sync_copy(acc_v, acc_hbm.at[wid])


def _edge_pass(q, kf, row_e, col_e, bias_e, dist_e, tocol, cpos_flat, rp):
    mesh = plsc.VectorSubcoreMesh(core_axis_name="c", subcore_axis_name="s")
    f = pl.kernel(
        _edge_body,
        mesh=mesh,
        out_type=jax.ShapeDtypeStruct((NW, ACC_PAD), jnp.float32),
        scratch_types=[
            pltpu.VMEM((QROWS, D), jnp.float32),
            pltpu.VMEM((M,), jnp.int32),
            pltpu.VMEM((3 * M,), jnp.float32),
            pltpu.VMEM((48,), jnp.int32),
            pltpu.VMEM((ACC_PAD,), jnp.float32),
            pltpu.VMEM((2 * E_BLK,), jnp.int32),
            pltpu.VMEM((2 * E_BLK,), jnp.int32),
            pltpu.VMEM((2 * E_BLK,), jnp.float32),
            pltpu.VMEM((2 * E_BLK,), jnp.float32),
            pltpu.VMEM((2 * K_BLK,), jnp.int32),
            pltpu.VMEM((2 * K_BLK, D), jnp.float32),
            pltpu.VMEM((32,), jnp.int32),
            pltpu.VMEM((16,), jnp.float32),
            pltpu.SemaphoreType.DMA,
            pltpu.SemaphoreType.DMA,
        ],
        compiler_params=pltpu.CompilerParams(needs_layout_passes=False),
    )
    return f(q, kf, row_e, col_e, bias_e, dist_e, tocol, cpos_flat, rp)


def _feat_body(acc_ref, pos_ref, fx_ref, fy_ref, fz_ref, av_ref):
    a = acc_ref[...]
    s = a[0:8, :]
    A = a[8:16, :]
    Dx = a[16:24, :]
    Dy = a[24:32, :]
    Dz = a[32:40, :]
    inv = jnp.where(s > 0.0, 1.0 / s, 0.0)
    avg = A * inv
    px = pos_ref[0:1, :]
    py = pos_ref[1:2, :]
    pz = pos_ref[2:3, :]
    dx = Dx * inv - avg * px
    dy = Dy * inv - avg * py
    dz = Dz * inv - avg * pz
    nrm = jnp.sqrt(dx * dx + dy * dy + dz * dz)
    sc = 1.0 / jnp.maximum(nrm, 1e-12)
    fx_ref[...] = dx * sc
    fy_ref[...] = dy * sc
    fz_ref[...] = dz * sc
    av_ref[...] = avg


def _featurize(acc_t, pos_t):
    B = 512
    grid = (N + B - 1) // B
    out = jax.ShapeDtypeStruct((H, N), jnp.float32)
    return pl.pallas_call(
        _feat_body,
        grid=(grid,),
        in_specs=[
            pl.BlockSpec((ACC_W, B), lambda i: (0, i)),
            pl.BlockSpec((3, B), lambda i: (0, i)),
        ],
        out_specs=[pl.BlockSpec((H, B), lambda i: (0, i))] * 4,
        out_shape=[out, out, out, out],
    )(acc_t, pos_t)


def kernel(x, row_index, col_index, to_col_index, att_bias, dist, pos,
           col_pos, Wq, bq, Wk, bk):
    x_pad = jnp.pad(x, ((0, N_PAD - N), (0, 0)))
    q, kf = _project(x_pad, Wq, bq, Wk, bk)

    row_i = row_index.astype(jnp.int32)
    rp = jnp.searchsorted(
        row_i, jnp.arange(33, dtype=jnp.int32) * RPW, side="left"
    ).astype(jnp.int32)
    rp = jnp.pad(rp, (0, 48 - 33), constant_values=E)

    row_e = jnp.pad(row_i, (0, E_PAD - E))
    col_e = jnp.pad(col_index.astype(jnp.int32), (0, E_PAD - E))
    bias_e = jnp.pad(att_bias, (0, E_PAD - E))
    dist_e = jnp.pad(dist, (0, E_PAD - E), constant_values=1.0)

    acc = _edge_pass(q, kf, row_e, col_e, bias_e, dist_e,
                     to_col_index.astype(jnp.int32),
                     col_pos.reshape(-1), rp)

    acc_t = acc[:, :RPW * ACC_W].reshape(NW * RPW, ACC_W)[:N].T
    fx, fy, fz, av = _featurize(acc_t, pos.T)

    feat = jnp.stack([fx, fy, fz, av], axis=-1)
    return feat.transpose(1, 0, 2).reshape(N, H * 4)


# speedup vs baseline: 41.5355x; 1.3007x over previous
import jax
import jax.numpy as jnp
from jax import lax
from jax.experimental import pallas as pl
from jax.experimental.pallas import tpu as pltpu
from jax.experimental.pallas import tpu_sc as plsc

N = 10000
E = 320000
M = 10000
D = 128
H = 8
DH = 16

NW = 32
RPW = 320
N_PAD = 10240
QW = 384
E_BLK = 256
K_BLK = 64
SPB = E_BLK // K_BLK
E_PAD = E + 2 * E_BLK
KTS = 65
ACC_W = 40
ACC_PAD = RPW * ACC_W

_GDN = lax.GatherDimensionNumbers(
    offset_dims=(), collapsed_slice_dims=(0,), start_index_map=(0,))


def _vgather(v, idx):
    return lax.gather(v, idx[:, None], _GDN, (1,),
                      mode=lax.GatherScatterMode.PROMISE_IN_BOUNDS)


def _proj_body(x_ref, wq_ref, bq_ref, wk_ref, bk_ref, qt_ref, kf_ref):
    xb = x_ref[...]
    q = jnp.dot(xb, wq_ref[...], preferred_element_type=jnp.float32)
    qt_ref[...] = ((q + bq_ref[...]) * 0.25).T
    kf = jnp.dot(xb, wk_ref[...], preferred_element_type=jnp.float32)
    kf_ref[...] = kf + bk_ref[...]


def _project(x_pad, Wq, bq, Wk, bk):
    grid = N_PAD // 256
    return pl.pallas_call(
        _proj_body,
        grid=(grid,),
        in_specs=[
            pl.BlockSpec((256, D), lambda i: (i, 0)),
            pl.BlockSpec((D, D), lambda i: (0, 0)),
            pl.BlockSpec((1, D), lambda i: (0, 0)),
            pl.BlockSpec((D, D), lambda i: (0, 0)),
            pl.BlockSpec((1, D), lambda i: (0, 0)),
        ],
        out_specs=[
            pl.BlockSpec((D, 256), lambda i: (0, i)),
            pl.BlockSpec((256, D), lambda i: (i, 0)),
        ],
        out_shape=[
            jax.ShapeDtypeStruct((D, N_PAD), jnp.float32),
            jax.ShapeDtypeStruct((N_PAD, D), jnp.float32),
        ],
    )(x_pad, Wq, bq.reshape(1, D), Wk, bk.reshape(1, D))


def _edge_body(qt_hbm, kf_hbm, row_hbm, col_hbm, bias_hbm, dist_hbm,
               tocol_hbm, cpos_hbm, rp_hbm, acc_hbm,
               qt_v, tocol_v, cpos_v, rp_v, acc_v,
               row_v, col_v, bias_v, dist_v, ci2_v, kbuf, kt_v,
               ksem, esem):
    wid = lax.axis_index("s") * 2 + lax.axis_index("c")
    rbase = wid * RPW
    rb128 = pl.multiple_of((rbase // 128) * 128, 128)
    rdelta = rbase - rb128
    iota = lax.iota(jnp.int32, 16)

    pltpu.sync_copy(rp_hbm, rp_v)
    pltpu.sync_copy(tocol_hbm, tocol_v)
    pltpu.sync_copy(cpos_hbm, cpos_v)
    pltpu.sync_copy(qt_hbm.at[:, pl.ds(rb128, QW)], qt_v)

    v0 = rp_v[pl.ds(0, 16)]
    v1 = rp_v[pl.ds(16, 16)]
    v2 = rp_v[pl.ds(32, 16)]

    def _ext(vec, j):
        picked = jnp.where(iota == j, vec.astype(jnp.float32), 0.0)
        return jnp.sum(picked).astype(jnp.int32)

    e_start = jnp.where(wid < 16, _ext(v0, wid), _ext(v1, wid - 16))
    w1 = wid + 1
    e_end = jnp.where(
        w1 < 16, _ext(v0, w1),
        jnp.where(w1 < 32, _ext(v1, w1 - 16), _ext(v2, w1 - 32)))

    zeros16 = jnp.zeros((16,), jnp.float32)

    def _z(i, c):
        acc_v[pl.ds(pl.multiple_of(i * 16, 16), 16)] = zeros16
        return c

    lax.fori_loop(0, ACC_PAD // 16, _z, 0)

    estart8 = pl.multiple_of((e_start // 8) * 8, 8)
    nsub = (e_end - estart8 + (K_BLK - 1)) // K_BLK
    nblk = (e_end - estart8 + (E_BLK - 1)) // E_BLK

    def _stage_start(b):
        bstart = pl.multiple_of(estart8 + b * E_BLK, 8)
        half = pl.multiple_of((b % 2) * E_BLK, 8)
        pltpu.async_copy(row_hbm.at[pl.ds(bstart, E_BLK)],
                         row_v.at[pl.ds(half, E_BLK)], esem)
        pltpu.async_copy(col_hbm.at[pl.ds(bstart, E_BLK)],
                         col_v.at[pl.ds(half, E_BLK)], esem)
        pltpu.async_copy(bias_hbm.at[pl.ds(bstart, E_BLK)],
                         bias_v.at[pl.ds(half, E_BLK)], esem)
        pltpu.async_copy(dist_hbm.at[pl.ds(bstart, E_BLK)],
                         dist_v.at[pl.ds(half, E_BLK)], esem)

    def _stage_wait(b):
        bstart = pl.multiple_of(estart8 + b * E_BLK, 8)
        half = pl.multiple_of((b % 2) * E_BLK, 8)
        pltpu.make_async_copy(row_hbm.at[pl.ds(bstart, E_BLK)],
                              row_v.at[pl.ds(half, E_BLK)], esem).wait()
        pltpu.make_async_copy(col_hbm.at[pl.ds(bstart, E_BLK)],
                              col_v.at[pl.ds(half, E_BLK)], esem).wait()
        pltpu.make_async_copy(bias_hbm.at[pl.ds(bstart, E_BLK)],
                              bias_v.at[pl.ds(half, E_BLK)], esem).wait()
        pltpu.make_async_copy(dist_hbm.at[pl.ds(bstart, E_BLK)],
                              dist_v.at[pl.ds(half, E_BLK)], esem).wait()

    def _kslices(s):
        khalf = pl.multiple_of((s % 2) * K_BLK, 8)
        return ci2_v.at[pl.ds(khalf, K_BLK)], kbuf.at[pl.ds(khalf, K_BLK)]

    def _ci2_and_fire(s):
        ehalf = ((s // SPB) % 2) * E_BLK
        ebase = ehalf + (s % SPB) * K_BLK
        khalf = (s % 2) * K_BLK
        for g in range(K_BLK // 16):
            off = pl.multiple_of(ebase + g * 16, 16)
            col16 = col_v[pl.ds(off, 16)]
            ci2 = plsc.load_gather(tocol_v, [col16])
            ci2_v[pl.ds(pl.multiple_of(khalf + g * 16, 16), 16)] = ci2
        ci_ref, kb_ref = _kslices(s)
        pltpu.async_copy(kf_hbm.at[ci_ref], kb_ref, ksem)

    def _kwait(s):
        ci_ref, kb_ref = _kslices(s)
        pltpu.make_async_copy(kf_hbm.at[ci_ref], kb_ref, ksem).wait()

    def _ktranspose(s):
        khalf = (s % 2) * K_BLK
        c65 = [(iota + rv * 16) * KTS for rv in range(8)]
        for e in range(K_BLK):
            r = khalf + e
            for rv in range(8):
                ve = kbuf[r, pl.ds(rv * 16, 16)]
                plsc.store_scatter(kt_v, [c65[rv] + e], ve)

    def _group(s, g):
        ehalf = ((s // SPB) % 2) * E_BLK
        ebase = ehalf + (s % SPB) * K_BLK
        off = pl.multiple_of(ebase + g * 16, 16)
        row16 = row_v[pl.ds(off, 16)]
        col16 = col_v[pl.ds(off, 16)]
        bias16 = bias_v[pl.ds(off, 16)]
        dist16 = dist_v[pl.ds(off, 16)]
        eidx = (estart8 + s * K_BLK + g * 16) + iota
        emask = (eidx >= e_start) & (eidx < e_end)
        lrow = jnp.clip(row16 - rbase, 0, RPW - 1)
        lrowq = lrow + rdelta
        biasm = jnp.where(emask, bias16, -100.0)

        invd = jnp.where(dist16 == 0.0, 0.0, 1.0 / dist16)
        col3 = col16 * 3
        cpx = plsc.load_gather(cpos_v, [col3])
        cpy = plsc.load_gather(cpos_v, [col3 + 1])
        cpz = plsc.load_gather(cpos_v, [col3 + 2])

        prv = _vgather(lrow, jnp.maximum(iota - 1, 0))
        nxt = _vgather(lrow, jnp.minimum(iota + 1, 15))
        is_first = (lrow != prv) | (iota == 0)
        is_last = (lrow != nxt) | (iota == 15)
        f = plsc.cummax(jnp.where(is_first, iota, 0))
        fm1 = jnp.maximum(f - 1, 0)
        hasp = f > 0
        lrow40 = lrow * ACC_W

        def _seg_add(v, offc):
            b = plsc.cumsum(v)
            pb = _vgather(b, fm1)
            tot = b - jnp.where(hasp, pb, 0.0)
            plsc.addupdate_scatter(acc_v, [lrow40 + offc], tot, mask=is_last)

        for h in range(H):
            lg = biasm
            for d in range(DH):
                c = h * DH + d
                qv = plsc.load_gather(qt_v, [jnp.full((16,), c, jnp.int32),
                                             lrowq])
                kv = kt_v[pl.ds(c * KTS + g * 16, 16)]
                lg = lg + qv * kv
            p = jnp.exp(lg)
            w = p * invd
            _seg_add(p, h)
            _seg_add(w, 8 + h)
            _seg_add(w * cpx, 16 + h)
            _seg_add(w * cpy, 24 + h)
            _seg_add(w * cpz, 32 + h)

    @pl.when(nsub > 0)
    def _():
        _stage_start(0)
        _stage_wait(0)
        _ci2_and_fire(0)

    def _sub(s, carry):
        @pl.when((s % SPB == 0) & (s // SPB + 1 < nblk))
        def _():
            _stage_start(s // SPB + 1)

        @pl.when((s + 1 < nsub) & ((s + 1) % SPB != 0))
        def _():
            _ci2_and_fire(s + 1)

        _kwait(s)
        _ktranspose(s)
        for g in range(K_BLK // 16):
            _group(s, g)

        @pl.when((s + 1 < nsub) & ((s + 1) % SPB == 0))
        def _():
            _stage_wait(s // SPB + 1)
            _ci2_and_fire(s + 1)

        return carry

    lax.fori_loop(0, nsub, _sub, 0)
    pltpu.sync_copy(acc_v, acc_hbm.at[wid])


def _edge_pass(qt, kf, row_e, col_e, bias_e, dist_e, tocol, cpos_flat, rp):
    mesh = plsc.VectorSubcoreMesh(core_axis_name="c", subcore_axis_name="s")
    f = pl.kernel(
        _edge_body,
        mesh=mesh,
        out_type=jax.ShapeDtypeStruct((NW, ACC_PAD), jnp.float32),
        scratch_types=[
            pltpu.VMEM((D, QW), jnp.float32),
            pltpu.VMEM((M,), jnp.int32),
            pltpu.VMEM((3 * M,), jnp.float32),
            pltpu.VMEM((48,), jnp.int32),
            pltpu.VMEM((ACC_PAD,), jnp.float32),
            pltpu.VMEM((2 * E_BLK,), jnp.int32),
            pltpu.VMEM((2 * E_BLK,), jnp.int32),
            pltpu.VMEM((2 * E_BLK,), jnp.float32),
            pltpu.VMEM((2 * E_BLK,), jnp.float32),
            pltpu.VMEM((2 * K_BLK,), jnp.int32),
            pltpu.VMEM((2 * K_BLK, D), jnp.float32),
            pltpu.VMEM((D * KTS,), jnp.float32),
            pltpu.SemaphoreType.DMA,
            pltpu.SemaphoreType.DMA,
        ],
        compiler_params=pltpu.CompilerParams(needs_layout_passes=False),
    )
    return f(qt, kf, row_e, col_e, bias_e, dist_e, tocol, cpos_flat, rp)


def _feat_body(acc_ref, pos_ref, fx_ref, fy_ref, fz_ref, av_ref):
    a = acc_ref[...]
    s = a[0:8, :]
    A = a[8:16, :]
    Dx = a[16:24, :]
    Dy = a[24:32, :]
    Dz = a[32:40, :]
    inv = jnp.where(s > 0.0, 1.0 / s, 0.0)
    avg = A * inv
    px = pos_ref[0:1, :]
    py = pos_ref[1:2, :]
    pz = pos_ref[2:3, :]
    dx = Dx * inv - avg * px
    dy = Dy * inv - avg * py
    dz = Dz * inv - avg * pz
    nrm = jnp.sqrt(dx * dx + dy * dy + dz * dz)
    sc = 1.0 / jnp.maximum(nrm, 1e-12)
    fx_ref[...] = dx * sc
    fy_ref[...] = dy * sc
    fz_ref[...] = dz * sc
    av_ref[...] = avg


def _featurize(acc_t, pos_t):
    B = 512
    grid = (N + B - 1) // B
    out = jax.ShapeDtypeStruct((H, N), jnp.float32)
    return pl.pallas_call(
        _feat_body,
        grid=(grid,),
        in_specs=[
            pl.BlockSpec((ACC_W, B), lambda i: (0, i)),
            pl.BlockSpec((3, B), lambda i: (0, i)),
        ],
        out_specs=[pl.BlockSpec((H, B), lambda i: (0, i))] * 4,
        out_shape=[out, out, out, out],
    )(acc_t, pos_t)


def kernel(x, row_index, col_index, to_col_index, att_bias, dist, pos,
           col_pos, Wq, bq, Wk, bk):
    x_pad = jnp.pad(x, ((0, N_PAD - N), (0, 0)))
    qt, kf = _project(x_pad, Wq, bq, Wk, bk)

    row_i = row_index.astype(jnp.int32)
    rp = jnp.searchsorted(
        row_i, jnp.arange(33, dtype=jnp.int32) * RPW, side="left"
    ).astype(jnp.int32)
    rp = jnp.pad(rp, (0, 48 - 33), constant_values=E)

    row_e = jnp.pad(row_i, (0, E_PAD - E))
    col_e = jnp.pad(col_index.astype(jnp.int32), (0, E_PAD - E))
    bias_e = jnp.pad(att_bias, (0, E_PAD - E))
    dist_e = jnp.pad(dist, (0, E_PAD - E), constant_values=1.0)

    acc = _edge_pass(qt, kf, row_e, col_e, bias_e, dist_e,
                     to_col_index.astype(jnp.int32),
                     col_pos.reshape(-1), rp)

    acc_t = acc.reshape(NW * RPW, ACC_W)[:N].T
    fx, fy, fz, av = _featurize(acc_t, pos.T)

    feat = jnp.stack([fx, fy, fz, av], axis=-1)
    return feat.transpose(1, 0, 2).reshape(N, H * 4)
